# baseline (speedup 1.0000x reference)
import functools

import jax
import jax.numpy as jnp
from jax import lax
from jax.experimental import pallas as pl
from jax.experimental.pallas import tpu as pltpu
from jax.experimental.pallas import tpu_sc as plsc

_B, _H, _W, _C = 16, 32, 32, 256
_NC, _NS = 2, 16
_NW = _NC * _NS
_CPW = (2 * _C) // _NW
_L = 16
_TR = 40

_mesh = plsc.VectorSubcoreMesh(core_axis_name="c", subcore_axis_name="s")


@functools.partial(
    pl.kernel,
    out_type=(
        jax.ShapeDtypeStruct((_B, 2 * _C), jnp.float32),
        jax.ShapeDtypeStruct((_B, 2 * _C, _H, _W), jnp.float32),
    ),
    mesh=_mesh,
    compiler_params=pltpu.CompilerParams(
        use_tc_tiling_on_sc=False, needs_layout_passes=False),
    scratch_types=[
        pltpu.VMEM((_TR, _C), jnp.float32),
        pltpu.VMEM((_CPW, _H, _W), jnp.float32),
        pltpu.VMEM((8, _C), jnp.float32),
        pltpu.VMEM((_B, 2 * _C), jnp.float32),
        pltpu.SemaphoreType.DMA,
    ],
)
def _sc_kernel(row_hbm, col_hbm, pose_hbm, pemb_hbm, m_hbm,
               tbuf, piece, posebuf, pbuf, sem_m):
    cid = lax.axis_index("c")
    sid = lax.axis_index("s")
    wid = sid * _NC + cid
    is_col = wid < _NW // 2
    cbase = lax.rem(wid * _CPW, _C)

    @pl.when(is_col)
    def _():
        pltpu.sync_copy(col_hbm.at[pl.ds(0, _TR)], tbuf)

    @pl.when(jnp.logical_not(is_col))
    def _():
        pltpu.sync_copy(row_hbm.at[pl.ds(0, _TR)], tbuf)

    lanes = jnp.arange(_L, dtype=jnp.int32)

    @pl.when(is_col)
    def _():
        for ci in range(_CPW):
            cols = jnp.full((_L,), ci, jnp.int32) + cbase
            for xh in range(_W // _L):
                v = plsc.load_gather(tbuf, [lanes + (xh * _L + 1), cols])
                for y in range(_H):
                    piece[ci, y, pl.ds(xh * _L, _L)] = v

    @pl.when(jnp.logical_not(is_col))
    def _():
        for ci in range(_CPW):
            cols = jnp.full((_L,), ci, jnp.int32) + cbase
            for y in range(_H):
                v = plsc.load_gather(tbuf, [jnp.full((_L,), y + 1, jnp.int32), cols])
                for xh in range(_W // _L):
                    piece[ci, y, pl.ds(xh * _L, _L)] = v

    copies = [
        pltpu.async_copy(piece, m_hbm.at[b, pl.ds(wid * _CPW, _CPW)], sem_m)
        for b in range(_B)
    ]

    @pl.when(wid == 0)
    def _():
        pltpu.sync_copy(pose_hbm.at[pl.ds(0, 8)], posebuf)
        for k in range(2 * _C // _L):
            v = posebuf[0, pl.ds((k % (_C // _L)) * _L, _L)]
            for r in range(_B):
                pbuf[r, pl.ds(k * _L, _L)] = v
        pltpu.sync_copy(pbuf, pemb_hbm)

    for cp in copies:
        cp.wait()


def kernel(x, row_embed, col_embed, pose_token_embed):
    del x
    p_emb, m_emb = _sc_kernel(row_embed, col_embed, pose_token_embed)
    return (p_emb, m_emb)


# speedup vs baseline: 1.1368x; 1.1368x over previous
import functools

import jax
import jax.numpy as jnp
from jax import lax
from jax.experimental import pallas as pl
from jax.experimental.pallas import tpu as pltpu
from jax.experimental.pallas import tpu_sc as plsc

_B, _H, _W, _C = 16, 32, 32, 256
_NC, _NS = 2, 16
_NW = _NC * _NS
_CPW = (2 * _C) // _NW
_L = 16
_TR = 40

_mesh = plsc.VectorSubcoreMesh(core_axis_name="c", subcore_axis_name="s")


@functools.partial(
    pl.kernel,
    out_type=(
        jax.ShapeDtypeStruct((_B, 2 * _C), jnp.float32),
        jax.ShapeDtypeStruct((_B, 2 * _C, _H, _W), jnp.float32),
    ),
    mesh=_mesh,
    compiler_params=pltpu.CompilerParams(needs_layout_passes=False),
    scratch_types=[
        pltpu.VMEM((_TR, _C), jnp.float32),
        pltpu.VMEM((_CPW, _H, _W), jnp.float32),
        pltpu.VMEM((8, _C), jnp.float32),
        pltpu.VMEM((_B, 2 * _C), jnp.float32),
        pltpu.SemaphoreType.DMA,
    ],
)
def _sc_kernel(row_hbm, col_hbm, pose_hbm, pemb_hbm, m_hbm,
               tbuf, piece, posebuf, pbuf, sem_m):
    cid = lax.axis_index("c")
    sid = lax.axis_index("s")
    wid = sid * _NC + cid
    is_col = wid < _NW // 2
    cbase = lax.rem(wid * _CPW, _C)

    @pl.when(is_col)
    def _():
        pltpu.sync_copy(col_hbm.at[pl.ds(0, _TR)], tbuf)

    @pl.when(jnp.logical_not(is_col))
    def _():
        pltpu.sync_copy(row_hbm.at[pl.ds(0, _TR)], tbuf)

    lanes = jnp.arange(_L, dtype=jnp.int32)

    @pl.when(is_col)
    def _():
        for ci in range(_CPW):
            cols = jnp.full((_L,), ci, jnp.int32) + cbase
            for xh in range(_W // _L):
                v = plsc.load_gather(tbuf, [lanes + (xh * _L + 1), cols])
                for y in range(_H):
                    piece[ci, y, pl.ds(xh * _L, _L)] = v

    @pl.when(jnp.logical_not(is_col))
    def _():
        for ci in range(_CPW):
            cols = jnp.full((_L,), ci, jnp.int32) + cbase
            for y in range(_H):
                v = plsc.load_gather(tbuf, [jnp.full((_L,), y + 1, jnp.int32), cols])
                for xh in range(_W // _L):
                    piece[ci, y, pl.ds(xh * _L, _L)] = v

    copies = [
        pltpu.async_copy(piece, m_hbm.at[b, pl.ds(wid * _CPW, _CPW)], sem_m)
        for b in range(_B)
    ]

    @pl.when(wid == 0)
    def _():
        pltpu.sync_copy(pose_hbm.at[pl.ds(0, 8)], posebuf)
        for k in range(2 * _C // _L):
            v = posebuf[0, pl.ds((k % (_C // _L)) * _L, _L)]
            for r in range(_B):
                pbuf[r, pl.ds(k * _L, _L)] = v
        pltpu.sync_copy(pbuf, pemb_hbm)

    for cp in copies:
        cp.wait()


def kernel(x, row_embed, col_embed, pose_token_embed):
    del x
    p_emb, m_emb = _sc_kernel(row_embed, col_embed, pose_token_embed)
    return (p_emb, m_emb)
